# Initial kernel scaffold; baseline (speedup 1.0000x reference)
import jax
import jax.numpy as jnp
from jax.experimental import pallas as pl


def kernel(node_state, edge_src, edge_dst):
    raise NotImplementedError("write your pallas kernel here")


# speedup vs baseline: 5.5117x; 5.5117x over previous
import functools

import jax
import jax.numpy as jnp
from jax import lax
from jax.experimental import pallas as pl
from jax.experimental.pallas import tpu as pltpu
from jax.experimental.pallas import tpu_sc as plsc


def _gather_incident(node_state, edge_src, edge_dst, *, nw, chunks, g):
    n, d = node_state.shape
    e = edge_src.shape[0]
    per_w = e // nw
    mesh = plsc.VectorSubcoreMesh(core_axis_name="c", subcore_axis_name="s")

    @functools.partial(
        pl.kernel,
        mesh=mesh,
        out_type=jax.ShapeDtypeStruct((e, 2 * d), jnp.float32),
        scratch_types=[
            pltpu.VMEM((chunks, g), jnp.int32),
            pltpu.VMEM((chunks, g), jnp.int32),
            pltpu.VMEM((g, d), jnp.float32),
            pltpu.VMEM((g, d), jnp.float32),
            pltpu.SemaphoreType.DMA,
            pltpu.SemaphoreType.DMA,
        ],
    )
    def k(node_hbm, src_hbm, dst_hbm, out_hbm, sidx_v, didx_v, srows_v,
          drows_v, sem_a, sem_b):
        nc = 2
        wid = lax.axis_index("s") * nc + lax.axis_index("c")
        pltpu.sync_copy(src_hbm.at[wid], sidx_v)
        pltpu.sync_copy(dst_hbm.at[wid], didx_v)
        base = wid * per_w

        def step(i, carry):
            row0 = base + i * g
            ca = pltpu.async_copy(node_hbm.at[sidx_v.at[i]], srows_v, sem_a)
            cb = pltpu.async_copy(node_hbm.at[didx_v.at[i]], drows_v, sem_b)
            ca.wait()
            cb.wait()
            pltpu.sync_copy(srows_v, out_hbm.at[pl.ds(row0, g), pl.ds(0, d)])
            pltpu.sync_copy(drows_v, out_hbm.at[pl.ds(row0, g), pl.ds(d, d)])
            return carry

        lax.fori_loop(0, chunks, step, 0, unroll=False)

    src_r = edge_src.astype(jnp.int32).reshape(nw, chunks, g)
    dst_r = edge_dst.astype(jnp.int32).reshape(nw, chunks, g)
    return k(node_state, src_r, dst_r)


def kernel(node_state, edge_src, edge_dst):
    e = edge_src.shape[0]
    nw = 32
    g = 80
    assert e % (nw * g) == 0
    return _gather_incident(node_state, edge_src, edge_dst,
                            nw=nw, chunks=e // (nw * g), g=g)


# speedup vs baseline: 6.9992x; 1.2699x over previous
import functools

import jax
import jax.numpy as jnp
from jax import lax
from jax.experimental import pallas as pl
from jax.experimental.pallas import tpu as pltpu
from jax.experimental.pallas import tpu_sc as plsc


def _gather_incident(node_state, edge_src, edge_dst, *, nw, g, q):
    n, d = node_state.shape
    e = edge_src.shape[0]
    per_w = e // nw
    f = g * q
    chunks = per_w // g
    fills = (chunks // q) & ~1
    tail_chunks = chunks - fills * q
    mesh = plsc.VectorSubcoreMesh(core_axis_name="c", subcore_axis_name="s")

    @functools.partial(
        pl.kernel,
        mesh=mesh,
        out_type=jax.ShapeDtypeStruct((e, 2 * d), jnp.float32),
        scratch_types=[
            pltpu.VMEM((chunks, g), jnp.int32),
            pltpu.VMEM((chunks, g), jnp.int32),
            pltpu.VMEM((f, d), jnp.float32),
            pltpu.VMEM((f, d), jnp.float32),
            pltpu.VMEM((f, d), jnp.float32),
            pltpu.VMEM((f, d), jnp.float32),
            pltpu.SemaphoreType.DMA,
            pltpu.SemaphoreType.DMA,
        ],
    )
    def k(node_hbm, src_hbm, dst_hbm, out_hbm, sidx_v, didx_v,
          sbuf0, dbuf0, sbuf1, dbuf1, sem0, sem1):
        nc = 2
        wid = lax.axis_index("s") * nc + lax.axis_index("c")
        pltpu.sync_copy(src_hbm.at[wid], sidx_v)
        pltpu.sync_copy(dst_hbm.at[wid], didx_v)
        base = wid * per_w

        def fire(fill, nq, sbuf, dbuf, sem):
            for j in range(nq):
                c = fill * q + j
                pltpu.async_copy(node_hbm.at[sidx_v.at[c]],
                                 sbuf.at[pl.ds(j * g, g)], sem)
                pltpu.async_copy(node_hbm.at[didx_v.at[c]],
                                 dbuf.at[pl.ds(j * g, g)], sem)

        def drain(fill, nq, sbuf, dbuf, sem):
            for j in range(nq):
                c = fill * q + j
                pltpu.make_async_copy(node_hbm.at[sidx_v.at[c]],
                                      sbuf.at[pl.ds(j * g, g)], sem).wait()
                pltpu.make_async_copy(node_hbm.at[didx_v.at[c]],
                                      dbuf.at[pl.ds(j * g, g)], sem).wait()

        def write(fill, rows, sbuf, dbuf):
            r0 = base + fill * f
            pltpu.sync_copy(sbuf.at[pl.ds(0, rows)],
                            out_hbm.at[pl.ds(r0, rows), pl.ds(0, d)])
            pltpu.sync_copy(dbuf.at[pl.ds(0, rows)],
                            out_hbm.at[pl.ds(r0, rows), pl.ds(d, d)])

        fire(0, q, sbuf0, dbuf0, sem0)

        def body(i, carry):
            fire(2 * i + 1, q, sbuf1, dbuf1, sem1)
            drain(2 * i, q, sbuf0, dbuf0, sem0)
            write(2 * i, f, sbuf0, dbuf0)

            @pl.when(i != fills // 2 - 1)
            def _():
                fire(2 * i + 2, q, sbuf0, dbuf0, sem0)

            drain(2 * i + 1, q, sbuf1, dbuf1, sem1)
            write(2 * i + 1, f, sbuf1, dbuf1)
            return carry

        lax.fori_loop(0, fills // 2, body, 0, unroll=False)

        if tail_chunks:
            fire(fills, tail_chunks, sbuf0, dbuf0, sem0)
            drain(fills, tail_chunks, sbuf0, dbuf0, sem0)
            write(fills, tail_chunks * g, sbuf0, dbuf0)

    src_r = edge_src.astype(jnp.int32).reshape(nw, chunks, g)
    dst_r = edge_dst.astype(jnp.int32).reshape(nw, chunks, g)
    return k(node_state, src_r, dst_r)


def kernel(node_state, edge_src, edge_dst):
    e = edge_src.shape[0]
    nw = 32
    g = 40
    q = 2
    assert e % (nw * g) == 0
    return _gather_incident(node_state, edge_src, edge_dst, nw=nw, g=g, q=q)


# speedup vs baseline: 7.1320x; 1.0190x over previous
import functools

import jax
import jax.numpy as jnp
from jax import lax
from jax.experimental import pallas as pl
from jax.experimental.pallas import tpu as pltpu
from jax.experimental.pallas import tpu_sc as plsc


def _gather_incident(node_state, edge_src, edge_dst, *, nw, g, q):
    n, d = node_state.shape
    e = edge_src.shape[0]
    per_w = e // nw
    f = g * q
    chunks = per_w // g
    fills = (chunks // q) & ~1
    tail_chunks = chunks - fills * q
    mesh = plsc.VectorSubcoreMesh(core_axis_name="c", subcore_axis_name="s")

    @functools.partial(
        pl.kernel,
        mesh=mesh,
        out_type=jax.ShapeDtypeStruct((e, 2 * d), jnp.float32),
        scratch_types=[
            pltpu.VMEM((chunks, g), jnp.int32),
            pltpu.VMEM((chunks, g), jnp.int32),
            pltpu.VMEM((f, d), jnp.float32),
            pltpu.VMEM((f, d), jnp.float32),
            pltpu.VMEM((f, d), jnp.float32),
            pltpu.VMEM((f, d), jnp.float32),
            pltpu.SemaphoreType.DMA,
            pltpu.SemaphoreType.DMA,
        ],
    )
    def k(node_hbm, src_hbm, dst_hbm, out_hbm, sidx_v, didx_v,
          sbuf0, dbuf0, sbuf1, dbuf1, sem0, sem1):
        nc = 2
        wid = lax.axis_index("s") * nc + lax.axis_index("c")
        pltpu.sync_copy(src_hbm.at[wid], sidx_v)
        pltpu.sync_copy(dst_hbm.at[wid], didx_v)
        base = wid * per_w

        def fire(fill, nq, sbuf, dbuf, sem):
            for j in range(nq):
                c = fill * q + j
                pltpu.async_copy(node_hbm.at[sidx_v.at[c]],
                                 sbuf.at[pl.ds(j * g, g)], sem)
                pltpu.async_copy(node_hbm.at[didx_v.at[c]],
                                 dbuf.at[pl.ds(j * g, g)], sem)

        def drain(fill, nq, sbuf, dbuf, sem):
            for j in range(nq):
                c = fill * q + j
                pltpu.make_async_copy(node_hbm.at[sidx_v.at[c]],
                                      sbuf.at[pl.ds(j * g, g)], sem).wait()
                pltpu.make_async_copy(node_hbm.at[didx_v.at[c]],
                                      dbuf.at[pl.ds(j * g, g)], sem).wait()

        def write(fill, rows, sbuf, dbuf):
            r0 = base + fill * f
            pltpu.sync_copy(sbuf.at[pl.ds(0, rows)],
                            out_hbm.at[pl.ds(r0, rows), pl.ds(0, d)])
            pltpu.sync_copy(dbuf.at[pl.ds(0, rows)],
                            out_hbm.at[pl.ds(r0, rows), pl.ds(d, d)])

        fire(0, q, sbuf0, dbuf0, sem0)

        def body(i, carry):
            fire(2 * i + 1, q, sbuf1, dbuf1, sem1)
            drain(2 * i, q, sbuf0, dbuf0, sem0)
            write(2 * i, f, sbuf0, dbuf0)

            @pl.when(i != fills // 2 - 1)
            def _():
                fire(2 * i + 2, q, sbuf0, dbuf0, sem0)

            drain(2 * i + 1, q, sbuf1, dbuf1, sem1)
            write(2 * i + 1, f, sbuf1, dbuf1)
            return carry

        lax.fori_loop(0, fills // 2, body, 0, unroll=False)

        if tail_chunks:
            fire(fills, tail_chunks, sbuf0, dbuf0, sem0)
            drain(fills, tail_chunks, sbuf0, dbuf0, sem0)
            write(fills, tail_chunks * g, sbuf0, dbuf0)

    src_r = edge_src.astype(jnp.int32).reshape(nw, chunks, g)
    dst_r = edge_dst.astype(jnp.int32).reshape(nw, chunks, g)
    return k(node_state, src_r, dst_r)


def kernel(node_state, edge_src, edge_dst):
    e = edge_src.shape[0]
    nw = 32
    g = 80
    q = 1
    assert e % (nw * g) == 0
    return _gather_incident(node_state, edge_src, edge_dst, nw=nw, g=g, q=q)
